# Initial kernel scaffold; baseline (speedup 1.0000x reference)
import jax
import jax.numpy as jnp
from jax.experimental import pallas as pl


def kernel(x, edge_attr_down, edge_attr_mid, edge_attr_up, params, edge_index_down, edge_index_mid, edge_index_up, range_down, range_mid, range_up):
    raise NotImplementedError("write your pallas kernel here")


# speedup vs baseline: 1.3756x; 1.3756x over previous
import functools

import jax
import jax.numpy as jnp
import numpy as np
from jax.experimental import pallas as pl
from jax.experimental.pallas import tpu as pltpu
from jax.experimental.pallas import tpu_sc as plsc

_WIDTH = 32
_WPAD = 128
_LEVEL = 3
_KW = [128, 64, 32]
_DOWN = [24000, 6000, 1500]
_MID = [48000, 12000, 3000]
_UP = [6000, 1500]
_NN = 13125
_NROWS = 13440
_NPTS = 10000

_E_TRUE = [24000, 6000, 1500, 3000, 1500, 12000, 6000, 48000]
_LKW = [128, 64, 32, 32, 32, 64, 64, 128]
_HAS_MID = [False, False, False, True, False, True, False, True]

_CH = 2048
_CHUNKS = [-(-e // _CH) for e in _E_TRUE]
_SEC = [c * _CH for c in _CHUNKS]
_NE_PAD = sum(_SEC)
_NCHUNKS = sum(_CHUNKS)
_CSTART = np.cumsum([0] + _CHUNKS[:-1]).tolist()
_CEND = (np.cumsum(_CHUNKS) - 1).tolist()
_ESTART = np.cumsum([0] + _SEC[:-1]).tolist()
_VEND = [s + e for s, e in zip(_ESTART, _E_TRUE)]
_INV = [1.0 / (e * _WIDTH) for e in _E_TRUE]

_NC, _NS = 2, 16
_NW = _NC * _NS
_GCH = 128
_CPT = _NE_PAD // (_NW * _GCH)
_BPW = _CPT * _GCH

_SUM_S = np.zeros((1024, _WPAD), dtype=np.float32)
_SUM_S[np.arange(1024), np.arange(1024) // _WIDTH] = 1.0


def _relu(v):
    return jnp.maximum(v, 0.0)


def _dot(a, b):
    return jax.lax.dot_general(a, b, (((1,), (0,)), ((), ())),
                               preferred_element_type=jnp.float32)


def _wprep_body(wl_ref, bl_ref, s_ref, wo_ref, bo_ref):
    wo_ref[0] = _dot(wl_ref[0], s_ref[...])
    bo_ref[0] = _dot(bl_ref[0], s_ref[...])


def _wprep(wl_raw, bl_raw, s_mat):
    return pl.pallas_call(
        _wprep_body,
        grid=(8,),
        in_specs=[
            pl.BlockSpec((1, 128, 1024), lambda i: (i, 0, 0)),
            pl.BlockSpec((1, 1, 1024), lambda i: (i, 0, 0)),
            pl.BlockSpec((1024, _WPAD), lambda i: (0, 0)),
        ],
        out_specs=[
            pl.BlockSpec((1, 128, _WPAD), lambda i: (i, 0, 0)),
            pl.BlockSpec((1, 1, _WPAD), lambda i: (i, 0, 0)),
        ],
        out_shape=[
            jax.ShapeDtypeStruct((8, 128, _WPAD), jnp.float32),
            jax.ShapeDtypeStruct((8, 1, _WPAD), jnp.float32),
        ],
    )(wl_raw, bl_raw, s_mat)


def _h0_body(x_ref, w_ref, b_ref, o_ref):
    o_ref[...] = _dot(x_ref[...], w_ref[...]) + b_ref[...]


def _h0(x_pad, w_in, b_in):
    return pl.pallas_call(
        _h0_body,
        out_shape=jax.ShapeDtypeStruct((_NROWS, _WPAD), jnp.float32),
    )(x_pad, w_in, b_in)


def _gather_body(table_hbm, idx_hbm, out_hbm, idx_v, buf0, buf1, sem0, sem1):
    cid = jax.lax.axis_index("c")
    sid = jax.lax.axis_index("s")
    wid = sid * _NC + cid
    pltpu.sync_copy(idx_hbm.at[wid], idx_v)

    bufs = [buf0, buf1]
    sems = [sem0, sem1]

    def gcopy(c):
        return pltpu.make_async_copy(
            table_hbm.at[idx_v.at[c]], bufs[c % 2], sems[c % 2])

    gcopy(0).start()
    for c in range(1, _CPT + 1):
        if c < _CPT:
            gcopy(c).start()
        gcopy(c - 1).wait()
        pltpu.sync_copy(bufs[(c - 1) % 2],
                        out_hbm.at[pl.ds(wid * _BPW + (c - 1) * _GCH, _GCH)])


def _sc_gather(h0p, idx3):
    mesh = plsc.VectorSubcoreMesh(core_axis_name="c", subcore_axis_name="s")
    f = pl.kernel(
        _gather_body,
        out_type=jax.ShapeDtypeStruct((_NE_PAD, _WPAD), jnp.float32),
        mesh=mesh,
        scratch_types=[
            pltpu.VMEM((_CPT, _GCH), jnp.int32),
            pltpu.VMEM((_GCH, _WPAD), jnp.float32),
            pltpu.VMEM((_GCH, _WPAD), jnp.float32),
            pltpu.SemaphoreType.DMA,
            pltpu.SemaphoreType.DMA,
        ],
    )
    return f(h0p, idx3)


def _chain_body(ea_ref, g_ref, w0_ref, b0_ref, w1_ref, b1_ref, w2_ref, b2_ref,
                wl_ref, bl_ref, out_ref, acc_sm, s_sm):
    i = pl.program_id(0)
    lyr = jnp.int32(0)
    for b in _CSTART[1:]:
        lyr = lyr + (i >= b).astype(jnp.int32)

    @pl.when(i == 0)
    def _():
        acc_sm[0] = 0.0

    a = ea_ref[...]
    a = _relu(_dot(a, w0_ref[0]) + b0_ref[0])
    a = _relu(_dot(a, w1_ref[0]) + b1_ref[0])
    a = _relu(_dot(a, w2_ref[0]) + b2_ref[0])
    k = _dot(a, wl_ref[0]) + bl_ref[0]

    v = g_ref[...]
    for j in range(7):
        v = jnp.where(lyr > j, _relu(v + s_sm[j]), v)

    vend = jnp.int32(_VEND[0])
    for j in range(1, 8):
        vend = jnp.where(lyr == j, jnp.int32(_VEND[j]), vend)
    row = i * _CH + jax.lax.broadcasted_iota(jnp.int32, (_CH, _WPAD), 0)
    contrib = jnp.sum(jnp.where(row < vend, v * k, 0.0))

    acc = acc_sm[0] + contrib
    for j in range(8):
        @pl.when(i == _CEND[j])
        def _(j=j):
            sval = acc * _INV[j]
            s_sm[j] = sval
            out_ref[0, j] = sval
            acc_sm[0] = 0.0

    is_end = (i == _CEND[0])
    for j in range(1, 8):
        is_end = jnp.logical_or(is_end, i == _CEND[j])

    @pl.when(jnp.logical_not(is_end))
    def _():
        acc_sm[0] = acc


def _chain(ea_all, g_all, w0s, b0s, w1s, b1s, w2s, b2s, wls, bls):
    def lmap(i):
        lyr = jnp.int32(0)
        for b in _CSTART[1:]:
            lyr = lyr + (i >= b).astype(jnp.int32)
        return lyr

    w3 = lambda d0, d1: pl.BlockSpec((1, d0, d1), lambda i: (lmap(i), 0, 0))
    return pl.pallas_call(
        _chain_body,
        grid=(_NCHUNKS,),
        in_specs=[
            pl.BlockSpec((_CH, 8), lambda i: (i, 0)),
            pl.BlockSpec((_CH, _WPAD), lambda i: (i, 0)),
            w3(8, 8), w3(1, 8), w3(8, 128), w3(1, 128),
            w3(128, 128), w3(1, 128), w3(128, _WPAD), w3(1, _WPAD),
        ],
        out_specs=pl.BlockSpec(memory_space=pltpu.SMEM),
        out_shape=jax.ShapeDtypeStruct((1, 8), jnp.float32),
        scratch_shapes=[
            pltpu.SMEM((1,), jnp.float32),
            pltpu.SMEM((8,), jnp.float32),
        ],
    )(ea_all, g_all, w0s, b0s, w1s, b1s, w2s, b2s, wls, bls)


def _head_body(h_ref, s_ref, w1_ref, b1_ref, w2_ref, b2_ref, o_ref):
    v = h_ref[...]
    for j in range(8):
        v = _relu(v + s_ref[0, j])
    y = _relu(_dot(v, w1_ref[...]) + b1_ref[...])
    o_ref[...] = (jnp.sum(y * w2_ref[...], axis=1, keepdims=True)
                  + b2_ref[0, 0])


def _head(h0p, s8, w1, b1, w2row, b2):
    return pl.pallas_call(
        _head_body,
        grid=(10,),
        in_specs=[
            pl.BlockSpec((1000, _WPAD), lambda i: (i, 0)),
            pl.BlockSpec(memory_space=pltpu.SMEM),
            pl.BlockSpec((_WPAD, 256), lambda i: (0, 0)),
            pl.BlockSpec((1, 256), lambda i: (0, 0)),
            pl.BlockSpec((1, 256), lambda i: (0, 0)),
            pl.BlockSpec(memory_space=pltpu.SMEM),
        ],
        out_specs=pl.BlockSpec((1000, 1), lambda i: (i, 0)),
        out_shape=jax.ShapeDtypeStruct((_NPTS, 1), jnp.float32),
    )(h0p, s8, w1, b1, w2row, b2)


def _layer_params(params):
    dk, mk, uk = params["down_k"], params["mid_k"], params["up_k"]
    return [dk[0], dk[1], dk[2], mk[2], uk[1], mk[1], uk[0], mk[0]]


def _sections(edge_attr_down, edge_attr_mid, edge_attr_up,
              edge_index_down, edge_index_mid, edge_index_up):
    d0 = np.cumsum([0] + _DOWN[:-1]).tolist()
    m0 = np.cumsum([0] + _MID[:-1]).tolist()
    u0 = np.cumsum([0] + _UP[:-1]).tolist()
    spec = [
        (edge_attr_down, edge_index_down, d0[0], _DOWN[0]),
        (edge_attr_down, edge_index_down, d0[1], _DOWN[1]),
        (edge_attr_down, edge_index_down, d0[2], _DOWN[2]),
        (edge_attr_mid, edge_index_mid, m0[2], _MID[2]),
        (edge_attr_up, edge_index_up, u0[1], _UP[1]),
        (edge_attr_mid, edge_index_mid, m0[1], _MID[1]),
        (edge_attr_up, edge_index_up, u0[0], _UP[0]),
        (edge_attr_mid, edge_index_mid, m0[0], _MID[0]),
    ]
    eas, idxs = [], []
    for (ea, ei, s, n), sec in zip(spec, _SEC):
        eas.append(jnp.pad(ea[s:s + n], ((0, sec - n), (0, 2))))
        idxs.append(jnp.pad(ei[1, s:s + n], (0, sec - n)))
    return jnp.concatenate(eas, axis=0), jnp.concatenate(idxs, axis=0)


def _stack_weights(params):
    lps = _layer_params(params)
    w0s = np.zeros((8, 8, 8), np.float32)
    b0s = np.zeros((8, 1, 8), np.float32)
    w1s = np.zeros((8, 8, 128), np.float32)
    b1s = np.zeros((8, 1, 128), np.float32)
    w2s = np.zeros((8, 128, 128), np.float32)
    b2s = np.zeros((8, 1, 128), np.float32)
    wlr = np.zeros((8, 128, 1024), np.float32)
    blr = np.zeros((8, 1, 1024), np.float32)
    w0s = jnp.asarray(w0s)
    b0s, w1s, b1s, w2s, b2s, wlr, blr = map(
        jnp.asarray, (b0s, w1s, b1s, w2s, b2s, wlr, blr))
    eye = jnp.eye(128, dtype=jnp.float32)
    for l, (p, kw, has_mid) in enumerate(zip(lps, _LKW, _HAS_MID)):
        W, b = p["W"], p["b"]
        w0s = w0s.at[l, :6, :6].set(W[0])
        b0s = b0s.at[l, 0, :6].set(b[0])
        w1s = w1s.at[l, :6, :kw].set(W[1])
        b1s = b1s.at[l, 0, :kw].set(b[1])
        if has_mid:
            w2s = w2s.at[l, :kw, :kw].set(W[2])
            b2s = b2s.at[l, 0, :kw].set(b[2])
        else:
            w2s = w2s.at[l].set(eye)
        wlr = wlr.at[l, :kw, :].set(W[-1])
        blr = blr.at[l, 0, :].set(b[-1])
    return w0s, b0s, w1s, b1s, w2s, b2s, wlr, blr


def kernel(x, edge_attr_down, edge_attr_mid, edge_attr_up, params,
           edge_index_down, edge_index_mid, edge_index_up,
           range_down, range_mid, range_up):
    del range_down, range_mid, range_up
    ea_all, src_all = _sections(edge_attr_down, edge_attr_mid, edge_attr_up,
                                edge_index_down, edge_index_mid, edge_index_up)
    idx3 = src_all.reshape(_NW, _CPT, _GCH)
    w0s, b0s, w1s, b1s, w2s, b2s, wlr, blr = _stack_weights(params)
    wls, bls = _wprep(wlr, blr, jnp.asarray(_SUM_S))

    x_pad = jnp.pad(x, ((0, _NROWS - _NN), (0, 2)))
    w_in = jnp.pad(params["mlp_in"]["W"][0], ((0, 2), (0, _WPAD - _WIDTH)))
    b_in = jnp.pad(params["mlp_in"]["b"][0].reshape(1, _WIDTH),
                   ((0, 0), (0, _WPAD - _WIDTH)))
    h0p = _h0(x_pad, w_in, b_in)

    g_all = _sc_gather(h0p, idx3)
    s8 = _chain(ea_all, g_all, w0s, b0s, w1s, b1s, w2s, b2s, wls, bls)

    w1 = jnp.pad(params["mlp_out1"]["W"][0], ((0, _WPAD - _WIDTH), (0, 0)))
    b1 = params["mlp_out1"]["b"][0].reshape(1, 256)
    w2row = params["mlp_out2"]["W"][0].reshape(1, 256)
    b2 = params["mlp_out2"]["b"][0].reshape(1, 1)
    return _head(h0p, s8, w1, b1, w2row, b2)


# speedup vs baseline: 1.3828x; 1.0053x over previous
import functools

import jax
import jax.numpy as jnp
import numpy as np
from jax.experimental import pallas as pl
from jax.experimental.pallas import tpu as pltpu
from jax.experimental.pallas import tpu_sc as plsc

_WIDTH = 32
_WPAD = 128
_LEVEL = 3
_KW = [128, 64, 32]
_DOWN = [24000, 6000, 1500]
_MID = [48000, 12000, 3000]
_UP = [6000, 1500]
_NN = 13125
_NROWS = 13440
_NPTS = 10000

_E_TRUE = [24000, 6000, 1500, 3000, 1500, 12000, 6000, 48000]
_LKW = [128, 64, 32, 32, 32, 64, 64, 128]
_HAS_MID = [False, False, False, True, False, True, False, True]

_CH = 2048
_CHUNKS = [-(-e // _CH) for e in _E_TRUE]
_SEC = [c * _CH for c in _CHUNKS]
_NE_PAD = sum(_SEC)
_NCHUNKS = sum(_CHUNKS)
_CSTART = np.cumsum([0] + _CHUNKS[:-1]).tolist()
_CEND = (np.cumsum(_CHUNKS) - 1).tolist()
_ESTART = np.cumsum([0] + _SEC[:-1]).tolist()
_VEND = [s + e for s, e in zip(_ESTART, _E_TRUE)]
_INV = [1.0 / (e * _WIDTH) for e in _E_TRUE]

_NC, _NS = 2, 16
_NW = _NC * _NS
_GCH = 128
_CPT = _NE_PAD // (_NW * _GCH)
_BPW = _CPT * _GCH

_SUM_S = np.repeat(np.eye(_WIDTH, dtype=np.float32), _WIDTH, axis=0)


def _relu(v):
    return jnp.maximum(v, 0.0)


def _dot(a, b):
    return jax.lax.dot_general(a, b, (((1,), (0,)), ((), ())),
                               preferred_element_type=jnp.float32)


def _wprep_body(wl_ref, bl_ref, s_ref, wo_ref, bo_ref):
    wo_ref[0] = _dot(wl_ref[0], s_ref[...])
    bo_ref[0] = _dot(bl_ref[0], s_ref[...])


def _wprep(wl_raw, bl_raw, s_mat):
    return pl.pallas_call(
        _wprep_body,
        grid=(8,),
        in_specs=[
            pl.BlockSpec((1, 128, 1024), lambda i: (i, 0, 0)),
            pl.BlockSpec((1, 1, 1024), lambda i: (i, 0, 0)),
            pl.BlockSpec((1024, _WIDTH), lambda i: (0, 0)),
        ],
        out_specs=[
            pl.BlockSpec((1, 128, _WIDTH), lambda i: (i, 0, 0)),
            pl.BlockSpec((1, 1, _WIDTH), lambda i: (i, 0, 0)),
        ],
        out_shape=[
            jax.ShapeDtypeStruct((8, 128, _WIDTH), jnp.float32),
            jax.ShapeDtypeStruct((8, 1, _WIDTH), jnp.float32),
        ],
    )(wl_raw, bl_raw, s_mat)


def _h0_body(x_ref, w_ref, b_ref, o_ref):
    o_ref[...] = _dot(x_ref[...], w_ref[...]) + b_ref[...]


def _h0(x_pad, w_in, b_in):
    return pl.pallas_call(
        _h0_body,
        out_shape=jax.ShapeDtypeStruct((_NROWS, _WPAD), jnp.float32),
    )(x_pad, w_in, b_in)


_NBUF = 6
_GLA = 3


def _gather_body(table_hbm, idx_hbm, out_hbm, idx_v, *bufsems):
    bufs = bufsems[:_NBUF]
    gsems = bufsems[_NBUF:2 * _NBUF]
    wsems = bufsems[2 * _NBUF:]
    cid = jax.lax.axis_index("c")
    sid = jax.lax.axis_index("s")
    wid = sid * _NC + cid
    pltpu.sync_copy(idx_hbm.at[wid], idx_v)

    def gcopy(c):
        return pltpu.make_async_copy(
            table_hbm.at[idx_v.at[c]], bufs[c % _NBUF], gsems[c % _NBUF])

    def wcopy(c):
        return pltpu.make_async_copy(
            bufs[c % _NBUF],
            out_hbm.at[pl.ds(wid * _BPW + c * _GCH, _GCH)],
            wsems[c % _NBUF])

    for c in range(_CPT):
        if c >= _NBUF:
            wcopy(c - _NBUF).wait()
        gcopy(c).start()
        if c >= _GLA:
            gcopy(c - _GLA).wait()
            wcopy(c - _GLA).start()
    for c in range(_CPT - _GLA, _CPT):
        gcopy(c).wait()
        wcopy(c).start()
    for c in range(max(0, _CPT - _NBUF), _CPT):
        wcopy(c).wait()


def _sc_gather(h0p, idx3):
    mesh = plsc.VectorSubcoreMesh(core_axis_name="c", subcore_axis_name="s")
    f = pl.kernel(
        _gather_body,
        out_type=jax.ShapeDtypeStruct((_NE_PAD, _WPAD), jnp.float32),
        mesh=mesh,
        scratch_types=(
            [pltpu.VMEM((_CPT, _GCH), jnp.int32)]
            + [pltpu.VMEM((_GCH, _WPAD), jnp.float32)] * _NBUF
            + [pltpu.SemaphoreType.DMA] * (2 * _NBUF)
        ),
    )
    return f(h0p, idx3)


def _chain_body(ea_ref, g_ref, w0_ref, b0_ref, w1_ref, b1_ref, w2_ref, b2_ref,
                wl_ref, bl_ref, out_ref, acc_sm, s_sm):
    i = pl.program_id(0)
    lyr = jnp.int32(0)
    for b in _CSTART[1:]:
        lyr = lyr + (i >= b).astype(jnp.int32)

    @pl.when(i == 0)
    def _():
        acc_sm[0] = 0.0

    a = ea_ref[...]
    a = _relu(_dot(a, w0_ref[0]) + b0_ref[0])
    a = _relu(_dot(a, w1_ref[0]) + b1_ref[0])
    a = _relu(_dot(a, w2_ref[0]) + b2_ref[0])
    k = _dot(a, wl_ref[0]) + bl_ref[0]

    v = g_ref[:, :_WIDTH]
    for j in range(7):
        v = jnp.where(lyr > j, _relu(v + s_sm[j]), v)

    vend = jnp.int32(_VEND[0])
    for j in range(1, 8):
        vend = jnp.where(lyr == j, jnp.int32(_VEND[j]), vend)
    row = i * _CH + jax.lax.broadcasted_iota(jnp.int32, (_CH, _WIDTH), 0)
    contrib = jnp.sum(jnp.where(row < vend, v * k, 0.0))

    acc = acc_sm[0] + contrib
    for j in range(8):
        @pl.when(i == _CEND[j])
        def _(j=j):
            sval = acc * _INV[j]
            s_sm[j] = sval
            out_ref[0, j] = sval
            acc_sm[0] = 0.0

    is_end = (i == _CEND[0])
    for j in range(1, 8):
        is_end = jnp.logical_or(is_end, i == _CEND[j])

    @pl.when(jnp.logical_not(is_end))
    def _():
        acc_sm[0] = acc


def _chain(ea_all, g_all, w0s, b0s, w1s, b1s, w2s, b2s, wls, bls):
    def lmap(i):
        lyr = jnp.int32(0)
        for b in _CSTART[1:]:
            lyr = lyr + (i >= b).astype(jnp.int32)
        return lyr

    w3 = lambda d0, d1: pl.BlockSpec((1, d0, d1), lambda i: (lmap(i), 0, 0))
    return pl.pallas_call(
        _chain_body,
        grid=(_NCHUNKS,),
        in_specs=[
            pl.BlockSpec((_CH, 8), lambda i: (i, 0)),
            pl.BlockSpec((_CH, _WPAD), lambda i: (i, 0)),
            w3(8, 8), w3(1, 8), w3(8, 128), w3(1, 128),
            w3(128, 128), w3(1, 128), w3(128, _WIDTH), w3(1, _WIDTH),
        ],
        out_specs=pl.BlockSpec(memory_space=pltpu.SMEM),
        out_shape=jax.ShapeDtypeStruct((1, 8), jnp.float32),
        scratch_shapes=[
            pltpu.SMEM((1,), jnp.float32),
            pltpu.SMEM((8,), jnp.float32),
        ],
    )(ea_all, g_all, w0s, b0s, w1s, b1s, w2s, b2s, wls, bls)


def _head_body(h_ref, s_ref, w1_ref, b1_ref, w2_ref, b2_ref, o_ref):
    v = h_ref[...]
    for j in range(8):
        v = _relu(v + s_ref[0, j])
    y = _relu(_dot(v, w1_ref[...]) + b1_ref[...])
    o_ref[...] = (jnp.sum(y * w2_ref[...], axis=1, keepdims=True)
                  + b2_ref[0, 0])


def _head(h0p, s8, w1, b1, w2row, b2):
    return pl.pallas_call(
        _head_body,
        grid=(10,),
        in_specs=[
            pl.BlockSpec((1000, _WPAD), lambda i: (i, 0)),
            pl.BlockSpec(memory_space=pltpu.SMEM),
            pl.BlockSpec((_WPAD, 256), lambda i: (0, 0)),
            pl.BlockSpec((1, 256), lambda i: (0, 0)),
            pl.BlockSpec((1, 256), lambda i: (0, 0)),
            pl.BlockSpec(memory_space=pltpu.SMEM),
        ],
        out_specs=pl.BlockSpec((1000, 1), lambda i: (i, 0)),
        out_shape=jax.ShapeDtypeStruct((_NPTS, 1), jnp.float32),
    )(h0p, s8, w1, b1, w2row, b2)


def _layer_params(params):
    dk, mk, uk = params["down_k"], params["mid_k"], params["up_k"]
    return [dk[0], dk[1], dk[2], mk[2], uk[1], mk[1], uk[0], mk[0]]


def _sections(edge_attr_down, edge_attr_mid, edge_attr_up,
              edge_index_down, edge_index_mid, edge_index_up):
    d0 = np.cumsum([0] + _DOWN[:-1]).tolist()
    m0 = np.cumsum([0] + _MID[:-1]).tolist()
    u0 = np.cumsum([0] + _UP[:-1]).tolist()
    spec = [
        (edge_attr_down, edge_index_down, d0[0], _DOWN[0]),
        (edge_attr_down, edge_index_down, d0[1], _DOWN[1]),
        (edge_attr_down, edge_index_down, d0[2], _DOWN[2]),
        (edge_attr_mid, edge_index_mid, m0[2], _MID[2]),
        (edge_attr_up, edge_index_up, u0[1], _UP[1]),
        (edge_attr_mid, edge_index_mid, m0[1], _MID[1]),
        (edge_attr_up, edge_index_up, u0[0], _UP[0]),
        (edge_attr_mid, edge_index_mid, m0[0], _MID[0]),
    ]
    eas, idxs = [], []
    for (ea, ei, s, n), sec in zip(spec, _SEC):
        eas.append(jnp.pad(ea[s:s + n], ((0, sec - n), (0, 2))))
        idxs.append(jnp.pad(ei[1, s:s + n], (0, sec - n)))
    return jnp.concatenate(eas, axis=0), jnp.concatenate(idxs, axis=0)


def _stack_weights(params):
    lps = _layer_params(params)
    w0s = np.zeros((8, 8, 8), np.float32)
    b0s = np.zeros((8, 1, 8), np.float32)
    w1s = np.zeros((8, 8, 128), np.float32)
    b1s = np.zeros((8, 1, 128), np.float32)
    w2s = np.zeros((8, 128, 128), np.float32)
    b2s = np.zeros((8, 1, 128), np.float32)
    wlr = np.zeros((8, 128, 1024), np.float32)
    blr = np.zeros((8, 1, 1024), np.float32)
    w0s = jnp.asarray(w0s)
    b0s, w1s, b1s, w2s, b2s, wlr, blr = map(
        jnp.asarray, (b0s, w1s, b1s, w2s, b2s, wlr, blr))
    eye = jnp.eye(128, dtype=jnp.float32)
    for l, (p, kw, has_mid) in enumerate(zip(lps, _LKW, _HAS_MID)):
        W, b = p["W"], p["b"]
        w0s = w0s.at[l, :6, :6].set(W[0])
        b0s = b0s.at[l, 0, :6].set(b[0])
        w1s = w1s.at[l, :6, :kw].set(W[1])
        b1s = b1s.at[l, 0, :kw].set(b[1])
        if has_mid:
            w2s = w2s.at[l, :kw, :kw].set(W[2])
            b2s = b2s.at[l, 0, :kw].set(b[2])
        else:
            w2s = w2s.at[l].set(eye)
        wlr = wlr.at[l, :kw, :].set(W[-1])
        blr = blr.at[l, 0, :].set(b[-1])
    return w0s, b0s, w1s, b1s, w2s, b2s, wlr, blr


def kernel(x, edge_attr_down, edge_attr_mid, edge_attr_up, params,
           edge_index_down, edge_index_mid, edge_index_up,
           range_down, range_mid, range_up):
    del range_down, range_mid, range_up
    ea_all, src_all = _sections(edge_attr_down, edge_attr_mid, edge_attr_up,
                                edge_index_down, edge_index_mid, edge_index_up)
    idx3 = src_all.reshape(_NW, _CPT, _GCH)
    w0s, b0s, w1s, b1s, w2s, b2s, wlr, blr = _stack_weights(params)
    wls, bls = _wprep(wlr, blr, jnp.asarray(_SUM_S))

    x_pad = jnp.pad(x, ((0, _NROWS - _NN), (0, 2)))
    w_in = jnp.pad(params["mlp_in"]["W"][0], ((0, 2), (0, _WPAD - _WIDTH)))
    b_in = jnp.pad(params["mlp_in"]["b"][0].reshape(1, _WIDTH),
                   ((0, 0), (0, _WPAD - _WIDTH)))
    h0p = _h0(x_pad, w_in, b_in)

    g_all = _sc_gather(h0p, idx3)
    s8 = _chain(ea_all, g_all, w0s, b0s, w1s, b1s, w2s, b2s, wls, bls)

    w1 = jnp.pad(params["mlp_out1"]["W"][0], ((0, _WPAD - _WIDTH), (0, 0)))
    b1 = params["mlp_out1"]["b"][0].reshape(1, 256)
    w2row = params["mlp_out2"]["W"][0].reshape(1, 256)
    b2 = params["mlp_out2"]["b"][0].reshape(1, 1)
    return _head(h0p, s8, w1, b1, w2row, b2)


# speedup vs baseline: 1.3903x; 1.0054x over previous
import functools

import jax
import jax.numpy as jnp
import numpy as np
from jax.experimental import pallas as pl
from jax.experimental.pallas import tpu as pltpu
from jax.experimental.pallas import tpu_sc as plsc

_WIDTH = 32
_WPAD = 128
_LEVEL = 3
_KW = [128, 64, 32]
_DOWN = [24000, 6000, 1500]
_MID = [48000, 12000, 3000]
_UP = [6000, 1500]
_NN = 13125
_NROWS = 13440
_NPTS = 10000

_E_TRUE = [24000, 6000, 1500, 3000, 1500, 12000, 6000, 48000]
_LKW = [128, 64, 32, 32, 32, 64, 64, 128]
_HAS_MID = [False, False, False, True, False, True, False, True]

_CH = 2048
_CHUNKS = [-(-e // _CH) for e in _E_TRUE]
_SEC = [c * _CH for c in _CHUNKS]
_NE_PAD = sum(_SEC)
_NCHUNKS = sum(_CHUNKS)
_CSTART = np.cumsum([0] + _CHUNKS[:-1]).tolist()
_CEND = (np.cumsum(_CHUNKS) - 1).tolist()
_ESTART = np.cumsum([0] + _SEC[:-1]).tolist()
_VEND = [s + e for s, e in zip(_ESTART, _E_TRUE)]
_INV = [1.0 / (e * _WIDTH) for e in _E_TRUE]

_NC, _NS = 2, 16
_NW = _NC * _NS
_GCH = 128
_CPT = _NE_PAD // (_NW * _GCH)
_BPW = _CPT * _GCH

_SUM_S = np.zeros((1024, _WPAD), dtype=np.float32)
_SUM_S[:, :_WIDTH] = np.repeat(np.eye(_WIDTH, dtype=np.float32), _WIDTH, axis=0)


def _relu(v):
    return jnp.maximum(v, 0.0)


def _dot(a, b):
    return jax.lax.dot_general(a, b, (((1,), (0,)), ((), ())),
                               preferred_element_type=jnp.float32)


def _wprep_body(wl_ref, bl_ref, s_ref, wo_ref, bo_ref):
    wo_ref[0] = _dot(wl_ref[0], s_ref[...])
    bo_ref[0] = _dot(bl_ref[0], s_ref[...])


def _wprep(wl_raw, bl_raw, s_mat):
    return pl.pallas_call(
        _wprep_body,
        grid=(8,),
        in_specs=[
            pl.BlockSpec((1, 128, 1024), lambda i: (i, 0, 0)),
            pl.BlockSpec((1, 1, 1024), lambda i: (i, 0, 0)),
            pl.BlockSpec((1024, _WPAD), lambda i: (0, 0)),
        ],
        out_specs=[
            pl.BlockSpec((1, 128, _WPAD), lambda i: (i, 0, 0)),
            pl.BlockSpec((1, 1, _WPAD), lambda i: (i, 0, 0)),
        ],
        out_shape=[
            jax.ShapeDtypeStruct((8, 128, _WPAD), jnp.float32),
            jax.ShapeDtypeStruct((8, 1, _WPAD), jnp.float32),
        ],
    )(wl_raw, bl_raw, s_mat)


def _h0_body(x_ref, w_ref, b_ref, o_ref):
    o_ref[...] = _dot(x_ref[...], w_ref[...]) + b_ref[...]


def _h0(x_pad, w_in, b_in):
    return pl.pallas_call(
        _h0_body,
        out_shape=jax.ShapeDtypeStruct((_NROWS, _WPAD), jnp.float32),
    )(x_pad, w_in, b_in)


_NBUF = 6
_GLA = 3


def _gather_body(table_hbm, idx_hbm, out_hbm, idx_v, *bufsems):
    bufs = bufsems[:_NBUF]
    gsems = bufsems[_NBUF:2 * _NBUF]
    wsems = bufsems[2 * _NBUF:]
    cid = jax.lax.axis_index("c")
    sid = jax.lax.axis_index("s")
    wid = sid * _NC + cid
    pltpu.sync_copy(idx_hbm.at[wid], idx_v)

    def gcopy(c):
        return pltpu.make_async_copy(
            table_hbm.at[idx_v.at[c]], bufs[c % _NBUF], gsems[c % _NBUF])

    def wcopy(c):
        return pltpu.make_async_copy(
            bufs[c % _NBUF],
            out_hbm.at[pl.ds(wid * _BPW + c * _GCH, _GCH)],
            wsems[c % _NBUF])

    for c in range(_CPT):
        if c >= _NBUF:
            wcopy(c - _NBUF).wait()
        gcopy(c).start()
        if c >= _GLA:
            gcopy(c - _GLA).wait()
            wcopy(c - _GLA).start()
    for c in range(_CPT - _GLA, _CPT):
        gcopy(c).wait()
        wcopy(c).start()
    for c in range(max(0, _CPT - _NBUF), _CPT):
        wcopy(c).wait()


def _sc_gather(h0p, idx3):
    mesh = plsc.VectorSubcoreMesh(core_axis_name="c", subcore_axis_name="s")
    f = pl.kernel(
        _gather_body,
        out_type=jax.ShapeDtypeStruct((_NE_PAD, _WPAD), jnp.float32),
        mesh=mesh,
        scratch_types=(
            [pltpu.VMEM((_CPT, _GCH), jnp.int32)]
            + [pltpu.VMEM((_GCH, _WPAD), jnp.float32)] * _NBUF
            + [pltpu.SemaphoreType.DMA] * (2 * _NBUF)
        ),
    )
    return f(h0p, idx3)


def _chain_body(ea_ref, g_ref, w0_ref, b0_ref, w1_ref, b1_ref, w2_ref, b2_ref,
                wl_ref, bl_ref, out_ref, acc_sm, s_sm):
    i = pl.program_id(0)
    lyr = jnp.int32(0)
    for b in _CSTART[1:]:
        lyr = lyr + (i >= b).astype(jnp.int32)

    @pl.when(i == 0)
    def _():
        acc_sm[0] = 0.0

    a = ea_ref[...]
    a = _relu(_dot(a, w0_ref[0]) + b0_ref[0])
    a = _relu(_dot(a, w1_ref[0]) + b1_ref[0])
    a = _relu(_dot(a, w2_ref[0]) + b2_ref[0])
    k = _dot(a, wl_ref[0]) + bl_ref[0]

    v = g_ref[...]
    for j in range(7):
        v = jnp.where(lyr > j, _relu(v + s_sm[j]), v)

    vend = jnp.int32(_VEND[0])
    for j in range(1, 8):
        vend = jnp.where(lyr == j, jnp.int32(_VEND[j]), vend)
    row = i * _CH + jax.lax.broadcasted_iota(jnp.int32, (_CH, _WPAD), 0)
    contrib = jnp.sum(jnp.where(row < vend, v * k, 0.0))

    acc = acc_sm[0] + contrib
    for j in range(8):
        @pl.when(i == _CEND[j])
        def _(j=j):
            sval = acc * _INV[j]
            s_sm[j] = sval
            out_ref[0, j] = sval
            acc_sm[0] = 0.0

    is_end = (i == _CEND[0])
    for j in range(1, 8):
        is_end = jnp.logical_or(is_end, i == _CEND[j])

    @pl.when(jnp.logical_not(is_end))
    def _():
        acc_sm[0] = acc


def _chain(ea_all, g_all, w0s, b0s, w1s, b1s, w2s, b2s, wls, bls):
    def lmap(i):
        lyr = jnp.int32(0)
        for b in _CSTART[1:]:
            lyr = lyr + (i >= b).astype(jnp.int32)
        return lyr

    w3 = lambda d0, d1: pl.BlockSpec((1, d0, d1), lambda i: (lmap(i), 0, 0))
    return pl.pallas_call(
        _chain_body,
        grid=(_NCHUNKS,),
        in_specs=[
            pl.BlockSpec((_CH, 8), lambda i: (i, 0)),
            pl.BlockSpec((_CH, _WPAD), lambda i: (i, 0)),
            w3(8, 8), w3(1, 8), w3(8, 128), w3(1, 128),
            w3(128, 128), w3(1, 128), w3(128, _WPAD), w3(1, _WPAD),
        ],
        out_specs=pl.BlockSpec(memory_space=pltpu.SMEM),
        out_shape=jax.ShapeDtypeStruct((1, 8), jnp.float32),
        scratch_shapes=[
            pltpu.SMEM((1,), jnp.float32),
            pltpu.SMEM((8,), jnp.float32),
        ],
    )(ea_all, g_all, w0s, b0s, w1s, b1s, w2s, b2s, wls, bls)


def _head_body(h_ref, s_ref, w1_ref, b1_ref, w2_ref, b2_ref, o_ref):
    v = h_ref[...]
    for j in range(8):
        v = _relu(v + s_ref[0, j])
    y = _relu(_dot(v, w1_ref[...]) + b1_ref[...])
    o_ref[...] = (jnp.sum(y * w2_ref[...], axis=1, keepdims=True)
                  + b2_ref[0, 0])


def _head(h0p, s8, w1, b1, w2row, b2):
    return pl.pallas_call(
        _head_body,
        grid=(10,),
        in_specs=[
            pl.BlockSpec((1000, _WPAD), lambda i: (i, 0)),
            pl.BlockSpec(memory_space=pltpu.SMEM),
            pl.BlockSpec((_WPAD, 256), lambda i: (0, 0)),
            pl.BlockSpec((1, 256), lambda i: (0, 0)),
            pl.BlockSpec((1, 256), lambda i: (0, 0)),
            pl.BlockSpec(memory_space=pltpu.SMEM),
        ],
        out_specs=pl.BlockSpec((1000, 1), lambda i: (i, 0)),
        out_shape=jax.ShapeDtypeStruct((_NPTS, 1), jnp.float32),
    )(h0p, s8, w1, b1, w2row, b2)


def _layer_params(params):
    dk, mk, uk = params["down_k"], params["mid_k"], params["up_k"]
    return [dk[0], dk[1], dk[2], mk[2], uk[1], mk[1], uk[0], mk[0]]


def _sections(edge_attr_down, edge_attr_mid, edge_attr_up,
              edge_index_down, edge_index_mid, edge_index_up):
    d0 = np.cumsum([0] + _DOWN[:-1]).tolist()
    m0 = np.cumsum([0] + _MID[:-1]).tolist()
    u0 = np.cumsum([0] + _UP[:-1]).tolist()
    spec = [
        (edge_attr_down, edge_index_down, d0[0], _DOWN[0]),
        (edge_attr_down, edge_index_down, d0[1], _DOWN[1]),
        (edge_attr_down, edge_index_down, d0[2], _DOWN[2]),
        (edge_attr_mid, edge_index_mid, m0[2], _MID[2]),
        (edge_attr_up, edge_index_up, u0[1], _UP[1]),
        (edge_attr_mid, edge_index_mid, m0[1], _MID[1]),
        (edge_attr_up, edge_index_up, u0[0], _UP[0]),
        (edge_attr_mid, edge_index_mid, m0[0], _MID[0]),
    ]
    eas, idxs = [], []
    for (ea, ei, s, n), sec in zip(spec, _SEC):
        eas.append(jnp.pad(ea[s:s + n], ((0, sec - n), (0, 2))))
        idxs.append(jnp.pad(ei[1, s:s + n], (0, sec - n)))
    return jnp.concatenate(eas, axis=0), jnp.concatenate(idxs, axis=0)


def _stack_weights(params):
    lps = _layer_params(params)
    w0s = np.zeros((8, 8, 8), np.float32)
    b0s = np.zeros((8, 1, 8), np.float32)
    w1s = np.zeros((8, 8, 128), np.float32)
    b1s = np.zeros((8, 1, 128), np.float32)
    w2s = np.zeros((8, 128, 128), np.float32)
    b2s = np.zeros((8, 1, 128), np.float32)
    wlr = np.zeros((8, 128, 1024), np.float32)
    blr = np.zeros((8, 1, 1024), np.float32)
    w0s = jnp.asarray(w0s)
    b0s, w1s, b1s, w2s, b2s, wlr, blr = map(
        jnp.asarray, (b0s, w1s, b1s, w2s, b2s, wlr, blr))
    eye = jnp.eye(128, dtype=jnp.float32)
    for l, (p, kw, has_mid) in enumerate(zip(lps, _LKW, _HAS_MID)):
        W, b = p["W"], p["b"]
        w0s = w0s.at[l, :6, :6].set(W[0])
        b0s = b0s.at[l, 0, :6].set(b[0])
        w1s = w1s.at[l, :6, :kw].set(W[1])
        b1s = b1s.at[l, 0, :kw].set(b[1])
        if has_mid:
            w2s = w2s.at[l, :kw, :kw].set(W[2])
            b2s = b2s.at[l, 0, :kw].set(b[2])
        else:
            w2s = w2s.at[l].set(eye)
        wlr = wlr.at[l, :kw, :].set(W[-1])
        blr = blr.at[l, 0, :].set(b[-1])
    return w0s, b0s, w1s, b1s, w2s, b2s, wlr, blr


def kernel(x, edge_attr_down, edge_attr_mid, edge_attr_up, params,
           edge_index_down, edge_index_mid, edge_index_up,
           range_down, range_mid, range_up):
    del range_down, range_mid, range_up
    ea_all, src_all = _sections(edge_attr_down, edge_attr_mid, edge_attr_up,
                                edge_index_down, edge_index_mid, edge_index_up)
    idx3 = src_all.reshape(_NW, _CPT, _GCH)
    w0s, b0s, w1s, b1s, w2s, b2s, wlr, blr = _stack_weights(params)
    wls, bls = _wprep(wlr, blr, jnp.asarray(_SUM_S))

    x_pad = jnp.pad(x, ((0, _NROWS - _NN), (0, 2)))
    w_in = jnp.pad(params["mlp_in"]["W"][0], ((0, 2), (0, _WPAD - _WIDTH)))
    b_in = jnp.pad(params["mlp_in"]["b"][0].reshape(1, _WIDTH),
                   ((0, 0), (0, _WPAD - _WIDTH)))
    h0p = _h0(x_pad, w_in, b_in)

    g_all = _sc_gather(h0p, idx3)
    s8 = _chain(ea_all, g_all, w0s, b0s, w1s, b1s, w2s, b2s, wls, bls)

    w1 = jnp.pad(params["mlp_out1"]["W"][0], ((0, _WPAD - _WIDTH), (0, 0)))
    b1 = params["mlp_out1"]["b"][0].reshape(1, 256)
    w2row = params["mlp_out2"]["W"][0].reshape(1, 256)
    b2 = params["mlp_out2"]["b"][0].reshape(1, 1)
    return _head(h0p, s8, w1, b1, w2row, b2)


# speedup vs baseline: 1.4753x; 1.0611x over previous
import functools

import jax
import jax.numpy as jnp
import numpy as np
from jax.experimental import pallas as pl
from jax.experimental.pallas import tpu as pltpu
from jax.experimental.pallas import tpu_sc as plsc

_WIDTH = 32
_WPAD = 128
_LEVEL = 3
_KW = [128, 64, 32]
_DOWN = [24000, 6000, 1500]
_MID = [48000, 12000, 3000]
_UP = [6000, 1500]
_NN = 13125
_NROWS = 13440
_NPTS = 10000

_E_TRUE = [24000, 6000, 1500, 3000, 1500, 12000, 6000, 48000]
_LKW = [128, 64, 32, 32, 32, 64, 64, 128]
_HAS_MID = [False, False, False, True, False, True, False, True]

_CH = 2048
_CHUNKS = [-(-e // _CH) for e in _E_TRUE]
_SEC = [c * _CH for c in _CHUNKS]
_NE_PAD = sum(_SEC)
_NCHUNKS = sum(_CHUNKS)
_CSTART = np.cumsum([0] + _CHUNKS[:-1]).tolist()
_CEND = (np.cumsum(_CHUNKS) - 1).tolist()
_ESTART = np.cumsum([0] + _SEC[:-1]).tolist()
_VEND = [s + e for s, e in zip(_ESTART, _E_TRUE)]
_INV = [1.0 / (e * _WIDTH) for e in _E_TRUE]

_NC, _NS = 2, 16
_NW = _NC * _NS
_GCH = 128

_CHUNK_A = 28
_ROWS_A = _CHUNK_A * _CH
_ROWS_B = _NE_PAD - _ROWS_A
_CPT_A = _ROWS_A // (_NW * _GCH)
_CPT_B = _ROWS_B // (_NW * _GCH)

_SUM_S = np.zeros((1024, _WPAD), dtype=np.float32)
_SUM_S[:, :_WIDTH] = np.repeat(np.eye(_WIDTH, dtype=np.float32), _WIDTH, axis=0)


def _relu(v):
    return jnp.maximum(v, 0.0)


def _dot(a, b):
    return jax.lax.dot_general(a, b, (((1,), (0,)), ((), ())),
                               preferred_element_type=jnp.float32)


def _wprep_body(wl_ref, bl_ref, s_ref, wo_ref, bo_ref):
    wo_ref[0] = _dot(wl_ref[0], s_ref[...])
    bo_ref[0] = _dot(bl_ref[0], s_ref[...])


def _wprep(wl_raw, bl_raw, s_mat):
    return pl.pallas_call(
        _wprep_body,
        grid=(8,),
        in_specs=[
            pl.BlockSpec((1, 128, 1024), lambda i: (i, 0, 0)),
            pl.BlockSpec((1, 1, 1024), lambda i: (i, 0, 0)),
            pl.BlockSpec((1024, _WPAD), lambda i: (0, 0)),
        ],
        out_specs=[
            pl.BlockSpec((1, 128, _WPAD), lambda i: (i, 0, 0)),
            pl.BlockSpec((1, 1, _WPAD), lambda i: (i, 0, 0)),
        ],
        out_shape=[
            jax.ShapeDtypeStruct((8, 128, _WPAD), jnp.float32),
            jax.ShapeDtypeStruct((8, 1, _WPAD), jnp.float32),
        ],
    )(wl_raw, bl_raw, s_mat)


def _h0_body(x_ref, w_ref, b_ref, o_ref):
    o_ref[...] = _dot(x_ref[...], w_ref[...]) + b_ref[...]


def _h0(x_pad, w_in, b_in):
    return pl.pallas_call(
        _h0_body,
        out_shape=jax.ShapeDtypeStruct((_NROWS, _WPAD), jnp.float32),
    )(x_pad, w_in, b_in)


_NBUF = 6
_GLA = 3


def _gather_body(cpt, table_hbm, idx_hbm, out_hbm, idx_v, *bufsems):
    bufs = bufsems[:_NBUF]
    gsems = bufsems[_NBUF:2 * _NBUF]
    wsems = bufsems[2 * _NBUF:]
    cid = jax.lax.axis_index("c")
    sid = jax.lax.axis_index("s")
    wid = sid * _NC + cid
    pltpu.sync_copy(idx_hbm.at[wid], idx_v)

    def gcopy(c):
        return pltpu.make_async_copy(
            table_hbm.at[idx_v.at[c]], bufs[c % _NBUF], gsems[c % _NBUF])

    def wcopy(c):
        return pltpu.make_async_copy(
            bufs[c % _NBUF],
            out_hbm.at[pl.ds(wid * (cpt * _GCH) + c * _GCH, _GCH)],
            wsems[c % _NBUF])

    for c in range(cpt):
        if c >= _NBUF:
            wcopy(c - _NBUF).wait()
        gcopy(c).start()
        if c >= _GLA:
            gcopy(c - _GLA).wait()
            wcopy(c - _GLA).start()
    for c in range(max(0, cpt - _GLA), cpt):
        gcopy(c).wait()
        wcopy(c).start()
    for c in range(max(0, cpt - _NBUF), cpt):
        wcopy(c).wait()


def _sc_gather(h0p, idx3):
    cpt = idx3.shape[1]
    mesh = plsc.VectorSubcoreMesh(core_axis_name="c", subcore_axis_name="s")
    f = pl.kernel(
        functools.partial(_gather_body, cpt),
        out_type=jax.ShapeDtypeStruct((_NW * cpt * _GCH, _WPAD), jnp.float32),
        mesh=mesh,
        scratch_types=(
            [pltpu.VMEM((cpt, _GCH), jnp.int32)]
            + [pltpu.VMEM((_GCH, _WPAD), jnp.float32)] * _NBUF
            + [pltpu.SemaphoreType.DMA] * (2 * _NBUF)
        ),
    )
    return f(h0p, idx3)


def _chain_body(ea_ref, g_ref, w0_ref, b0_ref, w1_ref, b1_ref, w2_ref, b2_ref,
                wl_ref, bl_ref, out_ref, acc_sm, s_sm):
    i = pl.program_id(0)
    lyr = jnp.int32(0)
    for b in _CSTART[1:]:
        lyr = lyr + (i >= b).astype(jnp.int32)

    @pl.when(i == 0)
    def _():
        acc_sm[0] = 0.0
        out_ref[0, 7] = 0.0

    a = ea_ref[...]
    a = _relu(_dot(a, w0_ref[0]) + b0_ref[0])
    a = _relu(_dot(a, w1_ref[0]) + b1_ref[0])
    a = _relu(_dot(a, w2_ref[0]) + b2_ref[0])
    k = _dot(a, wl_ref[0]) + bl_ref[0]

    v = g_ref[...]
    for j in range(7):
        v = jnp.where(lyr > j, _relu(v + s_sm[j]), v)

    vend = jnp.int32(_VEND[0])
    for j in range(1, 8):
        vend = jnp.where(lyr == j, jnp.int32(_VEND[j]), vend)
    row = i * _CH + jax.lax.broadcasted_iota(jnp.int32, (_CH, _WPAD), 0)
    contrib = jnp.sum(jnp.where(row < vend, v * k, 0.0))

    acc = acc_sm[0] + contrib
    for j in range(8):
        @pl.when(i == _CEND[j])
        def _(j=j):
            sval = acc * _INV[j]
            s_sm[j] = sval
            out_ref[0, j] = sval
            acc_sm[0] = 0.0

    is_end = (i == _CEND[0])
    for j in range(1, 8):
        is_end = jnp.logical_or(is_end, i == _CEND[j])

    @pl.when(jnp.logical_not(is_end))
    def _():
        acc_sm[0] = acc


def _chain_a(ea_a, g_a, w0s, b0s, w1s, b1s, w2s, b2s, wls, bls):
    def lmap(i):
        lyr = jnp.int32(0)
        for b in _CSTART[1:]:
            lyr = lyr + (i >= b).astype(jnp.int32)
        return lyr

    w3 = lambda d0, d1: pl.BlockSpec((1, d0, d1), lambda i: (lmap(i), 0, 0))
    return pl.pallas_call(
        _chain_body,
        grid=(_CHUNK_A,),
        in_specs=[
            pl.BlockSpec((_CH, 8), lambda i: (i, 0)),
            pl.BlockSpec((_CH, _WPAD), lambda i: (i, 0)),
            w3(8, 8), w3(1, 8), w3(8, 128), w3(1, 128),
            w3(128, 128), w3(1, 128), w3(128, _WPAD), w3(1, _WPAD),
        ],
        out_specs=pl.BlockSpec(memory_space=pltpu.SMEM),
        out_shape=jax.ShapeDtypeStruct((1, 8), jnp.float32),
        scratch_shapes=[
            pltpu.SMEM((1,), jnp.float32),
            pltpu.SMEM((8,), jnp.float32),
        ],
    )(ea_a, g_a, w0s, b0s, w1s, b1s, w2s, b2s, wls, bls)


def _chainb_body(sin_ref, ea_ref, g_ref, w0_ref, b0_ref, w1_ref, b1_ref,
                 w2_ref, b2_ref, wl_ref, bl_ref, out_ref, acc_sm):
    i = pl.program_id(0)

    @pl.when(i == 0)
    def _():
        acc_sm[0] = 0.0

    a = ea_ref[...]
    a = _relu(_dot(a, w0_ref[0]) + b0_ref[0])
    a = _relu(_dot(a, w1_ref[0]) + b1_ref[0])
    a = _relu(_dot(a, w2_ref[0]) + b2_ref[0])
    k = _dot(a, wl_ref[0]) + bl_ref[0]

    v = g_ref[...]
    for j in range(7):
        v = _relu(v + sin_ref[0, j])

    row = i * _CH + jax.lax.broadcasted_iota(jnp.int32, (_CH, _WPAD), 0)
    contrib = jnp.sum(jnp.where(row < _E_TRUE[7], v * k, 0.0))
    acc = acc_sm[0] + contrib

    @pl.when(i == _CHUNKS[7] - 1)
    def _():
        for j in range(7):
            out_ref[0, j] = sin_ref[0, j]
        out_ref[0, 7] = acc * _INV[7]

    @pl.when(i < _CHUNKS[7] - 1)
    def _():
        acc_sm[0] = acc


def _chain_b(s_in, ea_b, g_b, w0s, b0s, w1s, b1s, w2s, b2s, wls, bls):
    w3 = lambda d0, d1: pl.BlockSpec((1, d0, d1), lambda i: (7, 0, 0))
    return pl.pallas_call(
        _chainb_body,
        grid=(_CHUNKS[7],),
        in_specs=[
            pl.BlockSpec(memory_space=pltpu.SMEM),
            pl.BlockSpec((_CH, 8), lambda i: (i, 0)),
            pl.BlockSpec((_CH, _WPAD), lambda i: (i, 0)),
            w3(8, 8), w3(1, 8), w3(8, 128), w3(1, 128),
            w3(128, 128), w3(1, 128), w3(128, _WPAD), w3(1, _WPAD),
        ],
        out_specs=pl.BlockSpec(memory_space=pltpu.SMEM),
        out_shape=jax.ShapeDtypeStruct((1, 8), jnp.float32),
        scratch_shapes=[pltpu.SMEM((1,), jnp.float32)],
    )(s_in, ea_b, g_b, w0s, b0s, w1s, b1s, w2s, b2s, wls, bls)


def _head_body(h_ref, s_ref, w1_ref, b1_ref, w2_ref, b2_ref, o_ref):
    v = h_ref[...]
    for j in range(8):
        v = _relu(v + s_ref[0, j])
    y = _relu(_dot(v, w1_ref[...]) + b1_ref[...])
    o_ref[...] = (jnp.sum(y * w2_ref[...], axis=1, keepdims=True)
                  + b2_ref[0, 0])


def _head(h0p, s8, w1, b1, w2row, b2):
    return pl.pallas_call(
        _head_body,
        grid=(10,),
        in_specs=[
            pl.BlockSpec((1000, _WPAD), lambda i: (i, 0)),
            pl.BlockSpec(memory_space=pltpu.SMEM),
            pl.BlockSpec((_WPAD, 256), lambda i: (0, 0)),
            pl.BlockSpec((1, 256), lambda i: (0, 0)),
            pl.BlockSpec((1, 256), lambda i: (0, 0)),
            pl.BlockSpec(memory_space=pltpu.SMEM),
        ],
        out_specs=pl.BlockSpec((1000, 1), lambda i: (i, 0)),
        out_shape=jax.ShapeDtypeStruct((_NPTS, 1), jnp.float32),
    )(h0p, s8, w1, b1, w2row, b2)


def _layer_params(params):
    dk, mk, uk = params["down_k"], params["mid_k"], params["up_k"]
    return [dk[0], dk[1], dk[2], mk[2], uk[1], mk[1], uk[0], mk[0]]


def _sections(edge_attr_down, edge_attr_mid, edge_attr_up,
              edge_index_down, edge_index_mid, edge_index_up):
    d0 = np.cumsum([0] + _DOWN[:-1]).tolist()
    m0 = np.cumsum([0] + _MID[:-1]).tolist()
    u0 = np.cumsum([0] + _UP[:-1]).tolist()
    spec = [
        (edge_attr_down, edge_index_down, d0[0], _DOWN[0]),
        (edge_attr_down, edge_index_down, d0[1], _DOWN[1]),
        (edge_attr_down, edge_index_down, d0[2], _DOWN[2]),
        (edge_attr_mid, edge_index_mid, m0[2], _MID[2]),
        (edge_attr_up, edge_index_up, u0[1], _UP[1]),
        (edge_attr_mid, edge_index_mid, m0[1], _MID[1]),
        (edge_attr_up, edge_index_up, u0[0], _UP[0]),
        (edge_attr_mid, edge_index_mid, m0[0], _MID[0]),
    ]
    eas, idxs = [], []
    for (ea, ei, s, n), sec in zip(spec, _SEC):
        eas.append(jnp.pad(ea[s:s + n], ((0, sec - n), (0, 2))))
        idxs.append(jnp.pad(ei[1, s:s + n], (0, sec - n)))
    return jnp.concatenate(eas, axis=0), jnp.concatenate(idxs, axis=0)


def _stack_weights(params):
    lps = _layer_params(params)
    w0s = np.zeros((8, 8, 8), np.float32)
    b0s = np.zeros((8, 1, 8), np.float32)
    w1s = np.zeros((8, 8, 128), np.float32)
    b1s = np.zeros((8, 1, 128), np.float32)
    w2s = np.zeros((8, 128, 128), np.float32)
    b2s = np.zeros((8, 1, 128), np.float32)
    wlr = np.zeros((8, 128, 1024), np.float32)
    blr = np.zeros((8, 1, 1024), np.float32)
    w0s = jnp.asarray(w0s)
    b0s, w1s, b1s, w2s, b2s, wlr, blr = map(
        jnp.asarray, (b0s, w1s, b1s, w2s, b2s, wlr, blr))
    eye = jnp.eye(128, dtype=jnp.float32)
    for l, (p, kw, has_mid) in enumerate(zip(lps, _LKW, _HAS_MID)):
        W, b = p["W"], p["b"]
        w0s = w0s.at[l, :6, :6].set(W[0])
        b0s = b0s.at[l, 0, :6].set(b[0])
        w1s = w1s.at[l, :6, :kw].set(W[1])
        b1s = b1s.at[l, 0, :kw].set(b[1])
        if has_mid:
            w2s = w2s.at[l, :kw, :kw].set(W[2])
            b2s = b2s.at[l, 0, :kw].set(b[2])
        else:
            w2s = w2s.at[l].set(eye)
        wlr = wlr.at[l, :kw, :].set(W[-1])
        blr = blr.at[l, 0, :].set(b[-1])
    return w0s, b0s, w1s, b1s, w2s, b2s, wlr, blr


def kernel(x, edge_attr_down, edge_attr_mid, edge_attr_up, params,
           edge_index_down, edge_index_mid, edge_index_up,
           range_down, range_mid, range_up):
    del range_down, range_mid, range_up
    ea_all, src_all = _sections(edge_attr_down, edge_attr_mid, edge_attr_up,
                                edge_index_down, edge_index_mid, edge_index_up)
    idx_a = src_all[:_ROWS_A].reshape(_NW, _CPT_A, _GCH)
    idx_b = src_all[_ROWS_A:].reshape(_NW, _CPT_B, _GCH)
    w0s, b0s, w1s, b1s, w2s, b2s, wlr, blr = _stack_weights(params)
    wls, bls = _wprep(wlr, blr, jnp.asarray(_SUM_S))

    x_pad = jnp.pad(x, ((0, _NROWS - _NN), (0, 2)))
    w_in = jnp.pad(params["mlp_in"]["W"][0], ((0, 2), (0, _WPAD - _WIDTH)))
    b_in = jnp.pad(params["mlp_in"]["b"][0].reshape(1, _WIDTH),
                   ((0, 0), (0, _WPAD - _WIDTH)))
    h0p = _h0(x_pad, w_in, b_in)

    g_a = _sc_gather(h0p, idx_a)
    g_b = _sc_gather(h0p, idx_b)
    s7 = _chain_a(ea_all[:_ROWS_A], g_a,
                  w0s, b0s, w1s, b1s, w2s, b2s, wls, bls)
    s8 = _chain_b(s7, ea_all[_ROWS_A:], g_b,
                  w0s, b0s, w1s, b1s, w2s, b2s, wls, bls)

    w1 = jnp.pad(params["mlp_out1"]["W"][0], ((0, _WPAD - _WIDTH), (0, 0)))
    b1 = params["mlp_out1"]["b"][0].reshape(1, 256)
    w2row = params["mlp_out2"]["W"][0].reshape(1, 256)
    b2 = params["mlp_out2"]["b"][0].reshape(1, 1)
    return _head(h0p, s8, w1, b1, w2row, b2)


# speedup vs baseline: 1.4848x; 1.0065x over previous
import functools

import jax
import jax.numpy as jnp
import numpy as np
from jax.experimental import pallas as pl
from jax.experimental.pallas import tpu as pltpu
from jax.experimental.pallas import tpu_sc as plsc

_WIDTH = 32
_WPAD = 128
_LEVEL = 3
_KW = [128, 64, 32]
_DOWN = [24000, 6000, 1500]
_MID = [48000, 12000, 3000]
_UP = [6000, 1500]
_NN = 13125
_NROWS = 13440
_NPTS = 10000

_E_TRUE = [24000, 6000, 1500, 3000, 1500, 12000, 6000, 48000]
_LKW = [128, 64, 32, 32, 32, 64, 64, 128]
_HAS_MID = [False, False, False, True, False, True, False, True]

_CH = 2048
_CHUNKS = [-(-e // _CH) for e in _E_TRUE]
_SEC = [c * _CH for c in _CHUNKS]
_NE_PAD = sum(_SEC)
_NCHUNKS = sum(_CHUNKS)
_CSTART = np.cumsum([0] + _CHUNKS[:-1]).tolist()
_CEND = (np.cumsum(_CHUNKS) - 1).tolist()
_ESTART = np.cumsum([0] + _SEC[:-1]).tolist()
_VEND = [s + e for s, e in zip(_ESTART, _E_TRUE)]
_INV = [1.0 / (e * _WIDTH) for e in _E_TRUE]

_NC, _NS = 2, 16
_NW = _NC * _NS
_GCH = 128

_SEGS = [0, 12, 28, _NCHUNKS]

_SUM_S = np.zeros((1024, _WPAD), dtype=np.float32)
_SUM_S[:, :_WIDTH] = np.repeat(np.eye(_WIDTH, dtype=np.float32), _WIDTH, axis=0)


def _relu(v):
    return jnp.maximum(v, 0.0)


def _dot(a, b):
    return jax.lax.dot_general(a, b, (((1,), (0,)), ((), ())),
                               preferred_element_type=jnp.float32)


def _wprep_body(wl_ref, bl_ref, s_ref, wo_ref, bo_ref):
    wo_ref[0] = _dot(wl_ref[0], s_ref[...])
    bo_ref[0] = _dot(bl_ref[0], s_ref[...])


def _wprep(wl_raw, bl_raw, s_mat):
    return pl.pallas_call(
        _wprep_body,
        grid=(8,),
        in_specs=[
            pl.BlockSpec((1, 128, 1024), lambda i: (i, 0, 0)),
            pl.BlockSpec((1, 1, 1024), lambda i: (i, 0, 0)),
            pl.BlockSpec((1024, _WPAD), lambda i: (0, 0)),
        ],
        out_specs=[
            pl.BlockSpec((1, 128, _WPAD), lambda i: (i, 0, 0)),
            pl.BlockSpec((1, 1, _WPAD), lambda i: (i, 0, 0)),
        ],
        out_shape=[
            jax.ShapeDtypeStruct((8, 128, _WPAD), jnp.float32),
            jax.ShapeDtypeStruct((8, 1, _WPAD), jnp.float32),
        ],
    )(wl_raw, bl_raw, s_mat)


def _h0_body(x_ref, w_ref, b_ref, o_ref):
    o_ref[...] = _dot(x_ref[...], w_ref[...]) + b_ref[...]


def _h0(x_pad, w_in, b_in):
    return pl.pallas_call(
        _h0_body,
        out_shape=jax.ShapeDtypeStruct((_NROWS, _WPAD), jnp.float32),
    )(x_pad, w_in, b_in)


_NBUF = 6
_GLA = 3


def _gather_body(cpt, table_hbm, idx_hbm, out_hbm, idx_v, *bufsems):
    bufs = bufsems[:_NBUF]
    gsems = bufsems[_NBUF:2 * _NBUF]
    wsems = bufsems[2 * _NBUF:]
    cid = jax.lax.axis_index("c")
    sid = jax.lax.axis_index("s")
    wid = sid * _NC + cid
    pltpu.sync_copy(idx_hbm.at[wid], idx_v)

    def gcopy(c):
        return pltpu.make_async_copy(
            table_hbm.at[idx_v.at[c]], bufs[c % _NBUF], gsems[c % _NBUF])

    def wcopy(c):
        return pltpu.make_async_copy(
            bufs[c % _NBUF],
            out_hbm.at[pl.ds(wid * (cpt * _GCH) + c * _GCH, _GCH)],
            wsems[c % _NBUF])

    for c in range(cpt):
        if c >= _NBUF:
            wcopy(c - _NBUF).wait()
        gcopy(c).start()
        if c >= _GLA:
            gcopy(c - _GLA).wait()
            wcopy(c - _GLA).start()
    for c in range(max(0, cpt - _GLA), cpt):
        gcopy(c).wait()
        wcopy(c).start()
    for c in range(max(0, cpt - _NBUF), cpt):
        wcopy(c).wait()


def _sc_gather(h0p, idx3):
    cpt = idx3.shape[1]
    mesh = plsc.VectorSubcoreMesh(core_axis_name="c", subcore_axis_name="s")
    f = pl.kernel(
        functools.partial(_gather_body, cpt),
        out_type=jax.ShapeDtypeStruct((_NW * cpt * _GCH, _WPAD), jnp.float32),
        mesh=mesh,
        scratch_types=(
            [pltpu.VMEM((cpt, _GCH), jnp.int32)]
            + [pltpu.VMEM((_GCH, _WPAD), jnp.float32)] * _NBUF
            + [pltpu.SemaphoreType.DMA] * (2 * _NBUF)
        ),
    )
    return f(h0p, idx3)


def _chain_body(lo, sin_ref, ea_ref, g_ref, w0_ref, b0_ref, w1_ref, b1_ref,
                w2_ref, b2_ref, wl_ref, bl_ref, out_ref, acc_sm, s_sm):
    i = pl.program_id(0)
    gi = i + lo
    lyr = jnp.int32(0)
    for b in _CSTART[1:]:
        lyr = lyr + (gi >= b).astype(jnp.int32)

    @pl.when(i == 0)
    def _():
        acc_sm[0] = 0.0
        for j in range(8):
            s_sm[j] = sin_ref[0, j]
            out_ref[0, j] = sin_ref[0, j]

    a = ea_ref[...]
    a = _relu(_dot(a, w0_ref[0]) + b0_ref[0])
    a = _relu(_dot(a, w1_ref[0]) + b1_ref[0])
    a = _relu(_dot(a, w2_ref[0]) + b2_ref[0])
    k = _dot(a, wl_ref[0]) + bl_ref[0]

    v = g_ref[...]
    for j in range(7):
        v = jnp.where(lyr > j, _relu(v + s_sm[j]), v)

    vend = jnp.int32(_VEND[0])
    for j in range(1, 8):
        vend = jnp.where(lyr == j, jnp.int32(_VEND[j]), vend)
    row = gi * _CH + jax.lax.broadcasted_iota(jnp.int32, (_CH, _WPAD), 0)
    contrib = jnp.sum(jnp.where(row < vend, v * k, 0.0))

    acc = acc_sm[0] + contrib
    for j in range(8):
        @pl.when(gi == _CEND[j])
        def _(j=j):
            sval = acc * _INV[j]
            s_sm[j] = sval
            out_ref[0, j] = sval
            acc_sm[0] = 0.0

    is_end = (gi == _CEND[0])
    for j in range(1, 8):
        is_end = jnp.logical_or(is_end, gi == _CEND[j])

    @pl.when(jnp.logical_not(is_end))
    def _():
        acc_sm[0] = acc


def _chain_seg(lo, hi, s_in, ea_seg, g_seg,
               w0s, b0s, w1s, b1s, w2s, b2s, wls, bls):
    def lmap(i):
        lyr = jnp.int32(0)
        for b in _CSTART[1:]:
            lyr = lyr + (i + lo >= b).astype(jnp.int32)
        return lyr

    w3 = lambda d0, d1: pl.BlockSpec((1, d0, d1), lambda i: (lmap(i), 0, 0))
    return pl.pallas_call(
        functools.partial(_chain_body, lo),
        grid=(hi - lo,),
        in_specs=[
            pl.BlockSpec(memory_space=pltpu.SMEM),
            pl.BlockSpec((_CH, 8), lambda i: (i, 0)),
            pl.BlockSpec((_CH, _WPAD), lambda i: (i, 0)),
            w3(8, 8), w3(1, 8), w3(8, 128), w3(1, 128),
            w3(128, 128), w3(1, 128), w3(128, _WPAD), w3(1, _WPAD),
        ],
        out_specs=pl.BlockSpec(memory_space=pltpu.SMEM),
        out_shape=jax.ShapeDtypeStruct((1, 8), jnp.float32),
        scratch_shapes=[
            pltpu.SMEM((1,), jnp.float32),
            pltpu.SMEM((8,), jnp.float32),
        ],
    )(s_in, ea_seg, g_seg, w0s, b0s, w1s, b1s, w2s, b2s, wls, bls)


def _head_body(h_ref, s_ref, w1_ref, b1_ref, w2_ref, b2_ref, o_ref):
    v = h_ref[...]
    for j in range(8):
        v = _relu(v + s_ref[0, j])
    y = _relu(_dot(v, w1_ref[...]) + b1_ref[...])
    o_ref[...] = (jnp.sum(y * w2_ref[...], axis=1, keepdims=True)
                  + b2_ref[0, 0])


def _head(h0p, s8, w1, b1, w2row, b2):
    return pl.pallas_call(
        _head_body,
        grid=(10,),
        in_specs=[
            pl.BlockSpec((1000, _WPAD), lambda i: (i, 0)),
            pl.BlockSpec(memory_space=pltpu.SMEM),
            pl.BlockSpec((_WPAD, 256), lambda i: (0, 0)),
            pl.BlockSpec((1, 256), lambda i: (0, 0)),
            pl.BlockSpec((1, 256), lambda i: (0, 0)),
            pl.BlockSpec(memory_space=pltpu.SMEM),
        ],
        out_specs=pl.BlockSpec((1000, 1), lambda i: (i, 0)),
        out_shape=jax.ShapeDtypeStruct((_NPTS, 1), jnp.float32),
    )(h0p, s8, w1, b1, w2row, b2)


def _layer_params(params):
    dk, mk, uk = params["down_k"], params["mid_k"], params["up_k"]
    return [dk[0], dk[1], dk[2], mk[2], uk[1], mk[1], uk[0], mk[0]]


def _sections(edge_attr_down, edge_attr_mid, edge_attr_up,
              edge_index_down, edge_index_mid, edge_index_up):
    d0 = np.cumsum([0] + _DOWN[:-1]).tolist()
    m0 = np.cumsum([0] + _MID[:-1]).tolist()
    u0 = np.cumsum([0] + _UP[:-1]).tolist()
    spec = [
        (edge_attr_down, edge_index_down, d0[0], _DOWN[0]),
        (edge_attr_down, edge_index_down, d0[1], _DOWN[1]),
        (edge_attr_down, edge_index_down, d0[2], _DOWN[2]),
        (edge_attr_mid, edge_index_mid, m0[2], _MID[2]),
        (edge_attr_up, edge_index_up, u0[1], _UP[1]),
        (edge_attr_mid, edge_index_mid, m0[1], _MID[1]),
        (edge_attr_up, edge_index_up, u0[0], _UP[0]),
        (edge_attr_mid, edge_index_mid, m0[0], _MID[0]),
    ]
    eas, idxs = [], []
    for (ea, ei, s, n), sec in zip(spec, _SEC):
        eas.append(jnp.pad(ea[s:s + n], ((0, sec - n), (0, 2))))
        idxs.append(jnp.pad(ei[1, s:s + n], (0, sec - n)))
    return jnp.concatenate(eas, axis=0), jnp.concatenate(idxs, axis=0)


def _stack_weights(params):
    lps = _layer_params(params)
    w0s = np.zeros((8, 8, 8), np.float32)
    b0s = np.zeros((8, 1, 8), np.float32)
    w1s = np.zeros((8, 8, 128), np.float32)
    b1s = np.zeros((8, 1, 128), np.float32)
    w2s = np.zeros((8, 128, 128), np.float32)
    b2s = np.zeros((8, 1, 128), np.float32)
    wlr = np.zeros((8, 128, 1024), np.float32)
    blr = np.zeros((8, 1, 1024), np.float32)
    w0s = jnp.asarray(w0s)
    b0s, w1s, b1s, w2s, b2s, wlr, blr = map(
        jnp.asarray, (b0s, w1s, b1s, w2s, b2s, wlr, blr))
    eye = jnp.eye(128, dtype=jnp.float32)
    for l, (p, kw, has_mid) in enumerate(zip(lps, _LKW, _HAS_MID)):
        W, b = p["W"], p["b"]
        w0s = w0s.at[l, :6, :6].set(W[0])
        b0s = b0s.at[l, 0, :6].set(b[0])
        w1s = w1s.at[l, :6, :kw].set(W[1])
        b1s = b1s.at[l, 0, :kw].set(b[1])
        if has_mid:
            w2s = w2s.at[l, :kw, :kw].set(W[2])
            b2s = b2s.at[l, 0, :kw].set(b[2])
        else:
            w2s = w2s.at[l].set(eye)
        wlr = wlr.at[l, :kw, :].set(W[-1])
        blr = blr.at[l, 0, :].set(b[-1])
    return w0s, b0s, w1s, b1s, w2s, b2s, wlr, blr


def kernel(x, edge_attr_down, edge_attr_mid, edge_attr_up, params,
           edge_index_down, edge_index_mid, edge_index_up,
           range_down, range_mid, range_up):
    del range_down, range_mid, range_up
    ea_all, src_all = _sections(edge_attr_down, edge_attr_mid, edge_attr_up,
                                edge_index_down, edge_index_mid, edge_index_up)
    w0s, b0s, w1s, b1s, w2s, b2s, wlr, blr = _stack_weights(params)
    wls, bls = _wprep(wlr, blr, jnp.asarray(_SUM_S))

    x_pad = jnp.pad(x, ((0, _NROWS - _NN), (0, 2)))
    w_in = jnp.pad(params["mlp_in"]["W"][0], ((0, 2), (0, _WPAD - _WIDTH)))
    b_in = jnp.pad(params["mlp_in"]["b"][0].reshape(1, _WIDTH),
                   ((0, 0), (0, _WPAD - _WIDTH)))
    h0p = _h0(x_pad, w_in, b_in)

    gs = []
    for lo, hi in zip(_SEGS[:-1], _SEGS[1:]):
        rows = (hi - lo) * _CH
        idx = src_all[lo * _CH:hi * _CH].reshape(
            _NW, rows // (_NW * _GCH), _GCH)
        gs.append(_sc_gather(h0p, idx))

    s = jnp.zeros((1, 8), jnp.float32)
    for (lo, hi), g_seg in zip(zip(_SEGS[:-1], _SEGS[1:]), gs):
        s = _chain_seg(lo, hi, s, ea_all[lo * _CH:hi * _CH], g_seg,
                       w0s, b0s, w1s, b1s, w2s, b2s, wls, bls)
    s8 = s

    w1 = jnp.pad(params["mlp_out1"]["W"][0], ((0, _WPAD - _WIDTH), (0, 0)))
    b1 = params["mlp_out1"]["b"][0].reshape(1, 256)
    w2row = params["mlp_out2"]["W"][0].reshape(1, 256)
    b2 = params["mlp_out2"]["b"][0].reshape(1, 1)
    return _head(h0p, s8, w1, b1, w2row, b2)


# speedup vs baseline: 1.4914x; 1.0044x over previous
import functools

import jax
import jax.numpy as jnp
import numpy as np
from jax.experimental import pallas as pl
from jax.experimental.pallas import tpu as pltpu
from jax.experimental.pallas import tpu_sc as plsc

_WIDTH = 32
_WPAD = 128
_LEVEL = 3
_KW = [128, 64, 32]
_DOWN = [24000, 6000, 1500]
_MID = [48000, 12000, 3000]
_UP = [6000, 1500]
_NN = 13125
_NROWS = 13440
_NPTS = 10000

_E_TRUE = [24000, 6000, 1500, 3000, 1500, 12000, 6000, 48000]
_LKW = [128, 64, 32, 32, 32, 64, 64, 128]
_HAS_MID = [False, False, False, True, False, True, False, True]

_CH = 2048
_CHUNKS = [-(-e // _CH) for e in _E_TRUE]
_SEC = [c * _CH for c in _CHUNKS]
_NE_PAD = sum(_SEC)
_NCHUNKS = sum(_CHUNKS)
_CSTART = np.cumsum([0] + _CHUNKS[:-1]).tolist()
_CEND = (np.cumsum(_CHUNKS) - 1).tolist()
_ESTART = np.cumsum([0] + _SEC[:-1]).tolist()
_VEND = [s + e for s, e in zip(_ESTART, _E_TRUE)]
_INV = [1.0 / (e * _WIDTH) for e in _E_TRUE]

_NC, _NS = 2, 16
_NW = _NC * _NS
_GCH = 128

_SEGS = [0, 12, 28, _NCHUNKS]

_SUM_S = np.zeros((1024, _WPAD), dtype=np.float32)
_SUM_S[:, :_WIDTH] = np.repeat(np.eye(_WIDTH, dtype=np.float32), _WIDTH, axis=0)


def _relu(v):
    return jnp.maximum(v, 0.0)


def _dot(a, b):
    return jax.lax.dot_general(a, b, (((1,), (0,)), ((), ())),
                               preferred_element_type=jnp.float32)


def _wprep_body(wl_ref, bl_ref, s_ref, wo_ref, bo_ref):
    wo_ref[0] = _dot(wl_ref[0], s_ref[...])
    bo_ref[0] = _dot(bl_ref[0], s_ref[...])


def _wprep(wl_raw, bl_raw, s_mat):
    return pl.pallas_call(
        _wprep_body,
        grid=(8,),
        in_specs=[
            pl.BlockSpec((1, 128, 1024), lambda i: (i, 0, 0)),
            pl.BlockSpec((1, 1, 1024), lambda i: (i, 0, 0)),
            pl.BlockSpec((1024, _WPAD), lambda i: (0, 0)),
        ],
        out_specs=[
            pl.BlockSpec((1, 128, _WPAD), lambda i: (i, 0, 0)),
            pl.BlockSpec((1, 1, _WPAD), lambda i: (i, 0, 0)),
        ],
        out_shape=[
            jax.ShapeDtypeStruct((8, 128, _WPAD), jnp.float32),
            jax.ShapeDtypeStruct((8, 1, _WPAD), jnp.float32),
        ],
    )(wl_raw, bl_raw, s_mat)


def _h0_body(x_ref, w_ref, b_ref, o_ref):
    o_ref[...] = _dot(x_ref[...], w_ref[...]) + b_ref[...]


def _h0(x_pad, w_in, b_in):
    return pl.pallas_call(
        _h0_body,
        out_shape=jax.ShapeDtypeStruct((_NROWS, _WPAD), jnp.float32),
    )(x_pad, w_in, b_in)


_NBUF = 6
_GLA = 3


def _gather_body(cpt, table_hbm, idx_hbm, out_hbm, idx_v, *bufsems):
    bufs = bufsems[:_NBUF]
    gsems = bufsems[_NBUF:2 * _NBUF]
    wsems = bufsems[2 * _NBUF:]
    cid = jax.lax.axis_index("c")
    sid = jax.lax.axis_index("s")
    wid = sid * _NC + cid
    pltpu.sync_copy(idx_hbm.at[wid], idx_v)

    def gcopy(c):
        return pltpu.make_async_copy(
            table_hbm.at[idx_v.at[c]], bufs[c % _NBUF], gsems[c % _NBUF])

    def wcopy(c):
        return pltpu.make_async_copy(
            bufs[c % _NBUF],
            out_hbm.at[pl.ds(wid * (cpt * _GCH) + c * _GCH, _GCH)],
            wsems[c % _NBUF])

    for c in range(cpt):
        if c >= _NBUF:
            wcopy(c - _NBUF).wait()
        gcopy(c).start()
        if c >= _GLA:
            gcopy(c - _GLA).wait()
            wcopy(c - _GLA).start()
    for c in range(max(0, cpt - _GLA), cpt):
        gcopy(c).wait()
        wcopy(c).start()
    for c in range(max(0, cpt - _NBUF), cpt):
        wcopy(c).wait()


def _sc_gather(h0p, idx3):
    cpt = idx3.shape[1]
    mesh = plsc.VectorSubcoreMesh(core_axis_name="c", subcore_axis_name="s")
    f = pl.kernel(
        functools.partial(_gather_body, cpt),
        out_type=jax.ShapeDtypeStruct((_NW * cpt * _GCH, _WPAD), jnp.float32),
        mesh=mesh,
        scratch_types=(
            [pltpu.VMEM((cpt, _GCH), jnp.int32)]
            + [pltpu.VMEM((_GCH, _WPAD), jnp.float32)] * _NBUF
            + [pltpu.SemaphoreType.DMA] * (2 * _NBUF)
        ),
    )
    return f(h0p, idx3)


def _chain_body(lo, sin_ref, ea_ref, g_ref, w0_ref, b0_ref, w1_ref, b1_ref,
                w2_ref, b2_ref, wl_ref, bl_ref, out_ref, acc_sm, s_sm):
    i = pl.program_id(0)
    gi = i + lo
    lyr = jnp.int32(0)
    for b in _CSTART[1:]:
        lyr = lyr + (gi >= b).astype(jnp.int32)

    @pl.when(i == 0)
    def _():
        acc_sm[...] = jnp.zeros((1, _WPAD), jnp.float32)
        for j in range(8):
            s_sm[j] = sin_ref[0, j]
            out_ref[0, j] = sin_ref[0, j]

    a = ea_ref[...]
    a = _relu(_dot(a, w0_ref[0]) + b0_ref[0])
    a = _relu(_dot(a, w1_ref[0]) + b1_ref[0])
    a = _relu(_dot(a, w2_ref[0]) + b2_ref[0])
    k = _dot(a, wl_ref[0]) + bl_ref[0]

    v = g_ref[...]
    for j in range(7):
        v = jnp.where(lyr > j, _relu(v + s_sm[j]), v)

    vend = jnp.int32(_VEND[0])
    for j in range(1, 8):
        vend = jnp.where(lyr == j, jnp.int32(_VEND[j]), vend)
    row = gi * _CH + jax.lax.broadcasted_iota(jnp.int32, (_CH, _WPAD), 0)
    psum = jnp.sum(jnp.where(row < vend, v * k, 0.0), axis=0, keepdims=True)

    acc = acc_sm[...] + psum
    for j in range(8):
        @pl.when(gi == _CEND[j])
        def _(j=j):
            sval = jnp.sum(acc) * _INV[j]
            s_sm[j] = sval
            out_ref[0, j] = sval
            acc_sm[...] = jnp.zeros((1, _WPAD), jnp.float32)

    is_end = (gi == _CEND[0])
    for j in range(1, 8):
        is_end = jnp.logical_or(is_end, gi == _CEND[j])

    @pl.when(jnp.logical_not(is_end))
    def _():
        acc_sm[...] = acc


def _chain_seg(lo, hi, s_in, ea_seg, g_seg,
               w0s, b0s, w1s, b1s, w2s, b2s, wls, bls):
    def lmap(i):
        lyr = jnp.int32(0)
        for b in _CSTART[1:]:
            lyr = lyr + (i + lo >= b).astype(jnp.int32)
        return lyr

    w3 = lambda d0, d1: pl.BlockSpec((1, d0, d1), lambda i: (lmap(i), 0, 0))
    return pl.pallas_call(
        functools.partial(_chain_body, lo),
        grid=(hi - lo,),
        in_specs=[
            pl.BlockSpec(memory_space=pltpu.SMEM),
            pl.BlockSpec((_CH, 8), lambda i: (i, 0)),
            pl.BlockSpec((_CH, _WPAD), lambda i: (i, 0)),
            w3(8, 8), w3(1, 8), w3(8, 128), w3(1, 128),
            w3(128, 128), w3(1, 128), w3(128, _WPAD), w3(1, _WPAD),
        ],
        out_specs=pl.BlockSpec(memory_space=pltpu.SMEM),
        out_shape=jax.ShapeDtypeStruct((1, 8), jnp.float32),
        scratch_shapes=[
            pltpu.VMEM((1, _WPAD), jnp.float32),
            pltpu.SMEM((8,), jnp.float32),
        ],
    )(s_in, ea_seg, g_seg, w0s, b0s, w1s, b1s, w2s, b2s, wls, bls)


def _head_body(h_ref, s_ref, w1_ref, b1_ref, w2_ref, b2_ref, o_ref):
    v = h_ref[...]
    for j in range(8):
        v = _relu(v + s_ref[0, j])
    y = _relu(_dot(v, w1_ref[...]) + b1_ref[...])
    o_ref[...] = (jnp.sum(y * w2_ref[...], axis=1, keepdims=True)
                  + b2_ref[0, 0])


def _head(h0p, s8, w1, b1, w2row, b2):
    return pl.pallas_call(
        _head_body,
        grid=(10,),
        in_specs=[
            pl.BlockSpec((1000, _WPAD), lambda i: (i, 0)),
            pl.BlockSpec(memory_space=pltpu.SMEM),
            pl.BlockSpec((_WPAD, 256), lambda i: (0, 0)),
            pl.BlockSpec((1, 256), lambda i: (0, 0)),
            pl.BlockSpec((1, 256), lambda i: (0, 0)),
            pl.BlockSpec(memory_space=pltpu.SMEM),
        ],
        out_specs=pl.BlockSpec((1000, 1), lambda i: (i, 0)),
        out_shape=jax.ShapeDtypeStruct((_NPTS, 1), jnp.float32),
    )(h0p, s8, w1, b1, w2row, b2)


def _layer_params(params):
    dk, mk, uk = params["down_k"], params["mid_k"], params["up_k"]
    return [dk[0], dk[1], dk[2], mk[2], uk[1], mk[1], uk[0], mk[0]]


def _sections(edge_attr_down, edge_attr_mid, edge_attr_up,
              edge_index_down, edge_index_mid, edge_index_up):
    d0 = np.cumsum([0] + _DOWN[:-1]).tolist()
    m0 = np.cumsum([0] + _MID[:-1]).tolist()
    u0 = np.cumsum([0] + _UP[:-1]).tolist()
    spec = [
        (edge_attr_down, edge_index_down, d0[0], _DOWN[0]),
        (edge_attr_down, edge_index_down, d0[1], _DOWN[1]),
        (edge_attr_down, edge_index_down, d0[2], _DOWN[2]),
        (edge_attr_mid, edge_index_mid, m0[2], _MID[2]),
        (edge_attr_up, edge_index_up, u0[1], _UP[1]),
        (edge_attr_mid, edge_index_mid, m0[1], _MID[1]),
        (edge_attr_up, edge_index_up, u0[0], _UP[0]),
        (edge_attr_mid, edge_index_mid, m0[0], _MID[0]),
    ]
    eas, idxs = [], []
    for (ea, ei, s, n), sec in zip(spec, _SEC):
        eas.append(jnp.pad(ea[s:s + n], ((0, sec - n), (0, 2))))
        idxs.append(jnp.pad(ei[1, s:s + n], (0, sec - n)))
    return jnp.concatenate(eas, axis=0), jnp.concatenate(idxs, axis=0)


def _stack_weights(params):
    lps = _layer_params(params)
    w0s = np.zeros((8, 8, 8), np.float32)
    b0s = np.zeros((8, 1, 8), np.float32)
    w1s = np.zeros((8, 8, 128), np.float32)
    b1s = np.zeros((8, 1, 128), np.float32)
    w2s = np.zeros((8, 128, 128), np.float32)
    b2s = np.zeros((8, 1, 128), np.float32)
    wlr = np.zeros((8, 128, 1024), np.float32)
    blr = np.zeros((8, 1, 1024), np.float32)
    w0s = jnp.asarray(w0s)
    b0s, w1s, b1s, w2s, b2s, wlr, blr = map(
        jnp.asarray, (b0s, w1s, b1s, w2s, b2s, wlr, blr))
    eye = jnp.eye(128, dtype=jnp.float32)
    for l, (p, kw, has_mid) in enumerate(zip(lps, _LKW, _HAS_MID)):
        W, b = p["W"], p["b"]
        w0s = w0s.at[l, :6, :6].set(W[0])
        b0s = b0s.at[l, 0, :6].set(b[0])
        w1s = w1s.at[l, :6, :kw].set(W[1])
        b1s = b1s.at[l, 0, :kw].set(b[1])
        if has_mid:
            w2s = w2s.at[l, :kw, :kw].set(W[2])
            b2s = b2s.at[l, 0, :kw].set(b[2])
        else:
            w2s = w2s.at[l].set(eye)
        wlr = wlr.at[l, :kw, :].set(W[-1])
        blr = blr.at[l, 0, :].set(b[-1])
    return w0s, b0s, w1s, b1s, w2s, b2s, wlr, blr


def kernel(x, edge_attr_down, edge_attr_mid, edge_attr_up, params,
           edge_index_down, edge_index_mid, edge_index_up,
           range_down, range_mid, range_up):
    del range_down, range_mid, range_up
    ea_all, src_all = _sections(edge_attr_down, edge_attr_mid, edge_attr_up,
                                edge_index_down, edge_index_mid, edge_index_up)
    w0s, b0s, w1s, b1s, w2s, b2s, wlr, blr = _stack_weights(params)
    wls, bls = _wprep(wlr, blr, jnp.asarray(_SUM_S))

    x_pad = jnp.pad(x, ((0, _NROWS - _NN), (0, 2)))
    w_in = jnp.pad(params["mlp_in"]["W"][0], ((0, 2), (0, _WPAD - _WIDTH)))
    b_in = jnp.pad(params["mlp_in"]["b"][0].reshape(1, _WIDTH),
                   ((0, 0), (0, _WPAD - _WIDTH)))
    h0p = _h0(x_pad, w_in, b_in)

    gs = []
    for lo, hi in zip(_SEGS[:-1], _SEGS[1:]):
        rows = (hi - lo) * _CH
        idx = src_all[lo * _CH:hi * _CH].reshape(
            _NW, rows // (_NW * _GCH), _GCH)
        gs.append(_sc_gather(h0p, idx))

    s = jnp.zeros((1, 8), jnp.float32)
    for (lo, hi), g_seg in zip(zip(_SEGS[:-1], _SEGS[1:]), gs):
        s = _chain_seg(lo, hi, s, ea_all[lo * _CH:hi * _CH], g_seg,
                       w0s, b0s, w1s, b1s, w2s, b2s, wls, bls)
    s8 = s

    w1 = jnp.pad(params["mlp_out1"]["W"][0], ((0, _WPAD - _WIDTH), (0, 0)))
    b1 = params["mlp_out1"]["b"][0].reshape(1, 256)
    w2row = params["mlp_out2"]["W"][0].reshape(1, 256)
    b2 = params["mlp_out2"]["b"][0].reshape(1, 1)
    return _head(h0p, s8, w1, b1, w2row, b2)
